# baseline (speedup 1.0000x reference)
import functools

import jax
import jax.numpy as jnp
from jax import lax
from jax.experimental import pallas as pl
from jax.experimental.pallas import tpu as pltpu
from jax.experimental.pallas import tpu_sc as plsc

_ROWS = 16384
_COLS = 1024
_N = _ROWS * _COLS
_NW = 32
_PER_W = _N // _NW
_PIECE = 32768
_NPIECE = _PER_W // _PIECE
_VECS = _PIECE // 16
_UNROLL = 4

_C6 = -0.02512320328588397
_C5 = 0.11929823770562167
_C4 = -0.2746232576165834
_C3 = 0.45552708806081527
_C2 = -0.717557872422113
_C1 = 1.4424753148220728
_C0 = 2.123740891470091e-06
_LN2 = 0.6931471805599453


def _sc_yj(lm16, x_flat):
    mesh = plsc.VectorSubcoreMesh(core_axis_name="c", subcore_axis_name="s")

    @functools.partial(
        pl.kernel,
        mesh=mesh,
        out_type=jax.ShapeDtypeStruct((_N,), jnp.float32),
        scratch_types=[
            pltpu.VMEM((16,), jnp.float32),
            pltpu.VMEM((_PIECE,), jnp.float32),
            pltpu.VMEM((_PIECE,), jnp.float32),
        ],
    )
    def k(lm_hbm, x_hbm, o_hbm, lmv, xv, ov):
        wid = lax.axis_index("s") * 2 + lax.axis_index("c")
        pltpu.sync_copy(lm_hbm, lmv)
        lm = lmv[...]
        lme_p = jnp.maximum(lm, 1e-4)
        lme_n = 2.0 - lm
        m1 = 0.5 * (lme_p + lme_n)
        d1 = 0.5 * (lme_p - lme_n)
        inv_p = 1.0 / lme_p
        inv_n = 1.0 / lme_n
        m2 = 0.5 * (inv_p - inv_n)
        d2 = 0.5 * (inv_p + inv_n)
        d1b = lax.bitcast_convert_type(d1, jnp.int32)
        d2b = lax.bitcast_convert_type(d2, jnp.int32)
        base = wid * _PER_W

        def piece(j, carry):
            off = base + j * _PIECE
            pltpu.sync_copy(x_hbm.at[pl.ds(off, _PIECE)], xv)

            def vec(i, c2):
                for u_ in range(_UNROLL):
                    sl = pl.ds((i * _UNROLL + u_) * 16, 16)
                    xb = lax.bitcast_convert_type(xv[sl], jnp.int32)
                    sb = xb & jnp.int32(-0x80000000)
                    ax = lax.bitcast_convert_type(xb ^ sb, jnp.float32)
                    u = ax + 1.0
                    ub = lax.bitcast_convert_type(u, jnp.int32)
                    ef = lax.convert_element_type(
                        (ub >> 23) - 127, jnp.float32)
                    fm = lax.bitcast_convert_type(
                        (ub & jnp.int32(0x007FFFFF)) | jnp.int32(0x3F800000),
                        jnp.float32) - 1.0
                    poly = jnp.float32(_C6)
                    for c in (_C5, _C4, _C3, _C2, _C1, _C0):
                        poly = poly * fm + jnp.float32(c)
                    t2 = ef + poly
                    lme = m1 + lax.bitcast_convert_type(sb ^ d1b, jnp.float32)
                    p = jnp.exp((lme * _LN2) * t2)
                    sinv = m2 + lax.bitcast_convert_type(sb ^ d2b, jnp.float32)
                    ov[sl] = (p - 1.0) * sinv
                return c2

            lax.fori_loop(0, _VECS // _UNROLL, vec, 0)
            pltpu.sync_copy(ov, o_hbm.at[pl.ds(off, _PIECE)])
            return carry

        lax.fori_loop(0, _NPIECE, piece, 0)

    return k(lm16, x_flat)


def kernel(x, lmbda):
    lm16 = jnp.broadcast_to(lmbda, (16,))
    out = _sc_yj(lm16, x.reshape(-1))
    return out.reshape(_ROWS, _COLS)


# speedup vs baseline: 8.3221x; 8.3221x over previous
import jax
import jax.numpy as jnp
from jax import lax
from jax.experimental import pallas as pl
from jax.experimental.pallas import tpu as pltpu

_ROWS = 16384
_COLS = 1024
_BLOCK_ROWS = 2048
_CHUNK_ROWS = 1024


def _yj_body(lm_ref, x_ref, o_ref):
    lm = lm_ref[0]
    lme_p = jnp.maximum(lm, 1e-4)
    lme_n = 2.0 - lm
    inv_ln2 = 1.4426950408889634
    m1 = (0.5 * inv_ln2) * (lme_p + lme_n)
    d1 = (0.5 * inv_ln2) * (lme_p - lme_n)
    inv_p = 1.0 / lme_p
    inv_n = 1.0 / lme_n
    m2 = 0.5 * (inv_p - inv_n)
    d2 = 0.5 * (inv_p + inv_n)
    d1b = lax.bitcast_convert_type(d1, jnp.int32)
    d2b = lax.bitcast_convert_type(d2, jnp.int32)

    def chunk(i, _):
        rows = pl.ds(i * _CHUNK_ROWS, _CHUNK_ROWS)
        xb = lax.bitcast_convert_type(x_ref[rows, :], jnp.int32)
        ax = lax.bitcast_convert_type(xb & jnp.int32(0x7FFFFFFF), jnp.float32)
        sb = xb & jnp.int32(-0x80000000)
        t = jnp.log(ax + 1.0)
        lme = m1 + lax.bitcast_convert_type(sb ^ d1b, jnp.float32)
        p = lax.exp2(lme * t)
        sinv = m2 + lax.bitcast_convert_type(sb ^ d2b, jnp.float32)
        o_ref[rows, :] = (p - 1.0) * sinv
        return _

    lax.fori_loop(0, _BLOCK_ROWS // _CHUNK_ROWS, chunk, 0)


def kernel(x, lmbda):
    grid = (_ROWS // _BLOCK_ROWS,)
    return pl.pallas_call(
        _yj_body,
        grid=grid,
        in_specs=[
            pl.BlockSpec(memory_space=pltpu.SMEM),
            pl.BlockSpec((_BLOCK_ROWS, _COLS), lambda i: (i, 0)),
        ],
        out_specs=pl.BlockSpec((_BLOCK_ROWS, _COLS), lambda i: (i, 0)),
        out_shape=jax.ShapeDtypeStruct((_ROWS, _COLS), jnp.float32),
        compiler_params=pltpu.CompilerParams(
            dimension_semantics=("parallel",)),
    )(lmbda, x)


# speedup vs baseline: 8.3256x; 1.0004x over previous
import jax
import jax.numpy as jnp
from jax import lax
from jax.experimental import pallas as pl
from jax.experimental.pallas import tpu as pltpu

_ROWS = 16384
_COLS = 1024
_BLOCK_ROWS = 2048
_CHUNK_ROWS = 1024


def _yj_body(lm_ref, x_ref, o_ref):
    lm = lm_ref[0]
    lme_p = jnp.maximum(lm, 1e-4)
    lme_n = 2.0 - lm
    inv_ln2 = 1.4426950408889634
    m1 = (0.5 * inv_ln2) * (lme_p + lme_n)
    d1 = (0.5 * inv_ln2) * (lme_p - lme_n)
    inv_p = 1.0 / lme_p
    inv_n = 1.0 / lme_n
    m2 = 0.5 * (inv_p - inv_n)
    d2 = 0.5 * (inv_p + inv_n)
    d1b = lax.bitcast_convert_type(d1, jnp.int32)
    d2b = lax.bitcast_convert_type(d2, jnp.int32)

    def chunk(i, _):
        rows = pl.ds(i * _CHUNK_ROWS, _CHUNK_ROWS)
        xb = lax.bitcast_convert_type(x_ref[rows, :], jnp.int32)
        ax = lax.bitcast_convert_type(xb & jnp.int32(0x7FFFFFFF), jnp.float32)
        sb = xb & jnp.int32(-0x80000000)
        t = jnp.log(ax + 1.0)
        lme = m1 + lax.bitcast_convert_type(sb ^ d1b, jnp.float32)
        p = lax.exp2(lme * t)
        sinv = m2 + lax.bitcast_convert_type(sb ^ d2b, jnp.float32)
        o_ref[rows, :] = (p - 1.0) * sinv
        return _

    lax.fori_loop(0, _BLOCK_ROWS // _CHUNK_ROWS, chunk, 0)


def kernel(x, lmbda):
    grid = (_ROWS // _BLOCK_ROWS,)
    return pl.pallas_call(
        _yj_body,
        grid=grid,
        in_specs=[
            pl.BlockSpec(memory_space=pltpu.SMEM),
            pl.BlockSpec((_BLOCK_ROWS, _COLS), lambda i: (i, 0)),
        ],
        out_specs=pl.BlockSpec((_BLOCK_ROWS, _COLS), lambda i: (i, 0)),
        out_shape=jax.ShapeDtypeStruct((_ROWS, _COLS), jnp.float32),
    )(lmbda, x)
